# baseline (speedup 1.0000x reference)
import jax
import jax.numpy as jnp
from jax.experimental import pallas as pl
from jax.experimental.pallas import tpu as pltpu


def _smooth_l1(x):
    ax = jnp.abs(x)
    m = jnp.minimum(ax, 1.0)
    return m * ax - 0.5 * m * m


def _sublane_view(x):
    n, k = x.shape
    return jnp.swapaxes(x.reshape(n // 128, 128, k), 1, 2)


def _loss_kernel(
    lp_ref,
    lt_ref,
    cp_ref,
    t_ref,
    ign_ref,
    rlp_ref,
    rcp_ref,
    rlt_ref,
    rct_ref,
    out_ref,
    acc_ref,
):
    step = pl.program_id(0)
    nsteps = pl.num_programs(0)
    f32 = jnp.float32

    @pl.when(step == 0)
    def _init():
        acc_ref[...] = jnp.zeros_like(acc_ref)

    B = t_ref.shape[0]
    CH = 32
    sgn = jnp.where(
        jax.lax.broadcasted_iota(jnp.int32, (1, 2, 1), 1) == 1, 1.0, -1.0)
    run_slp = jnp.zeros((CH, 128), f32)
    run_ce = jnp.zeros((CH, 128), f32)
    run_pos = jnp.zeros((CH, 128), f32)
    for c in range(B // CH):
        s = pl.ds(c * CH, CH)
        slsum = jnp.sum(_smooth_l1(lp_ref[s] - lt_ref[s]), axis=1)
        d10 = jnp.sum(cp_ref[s] * sgn, axis=1)
        tc = t_ref[s]
        posf = (tc != 0).astype(f32)
        ce = jax.nn.softplus((1.0 - 2.0 * tc.astype(f32)) * d10)
        run_slp = run_slp + slsum * posf
        run_ce = run_ce + ce * (1.0 - ign_ref[s])
        run_pos = run_pos + posf

    acc_ref[0:1, :] += jnp.sum(run_slp, axis=0, keepdims=True)
    acc_ref[1:2, :] += jnp.sum(run_ce, axis=0, keepdims=True)
    acc_ref[2:3, :] += jnp.sum(run_pos, axis=0, keepdims=True)

    @pl.when(step == 0)
    def _rcnn():
        R = rlp_ref.shape[0]
        row_i = jax.lax.broadcasted_iota(jnp.int32, (R, 128), 0)
        lane_i = jax.lax.broadcasted_iota(jnp.int32, (R, 128), 1)
        P128 = (lane_i[:, 0:32] == row_i[:, 0:32] // 128).astype(f32)
        lane_sel = (lane_i == row_i % 128)

        ctx = jax.lax.dot_general(
            P128, rct_ref[...].astype(f32), (((1,), (0,)), ((), ())),
            preferred_element_type=f32)
        ct = jnp.sum(jnp.where(lane_sel, ctx, 0.0),
                     axis=1, keepdims=True)
        pos = jnp.where(ct != 0.0, 1.0, 0.0)
        idx = jnp.clip(ct - 1.0, 0.0, 79.0)

        lanes320 = jax.lax.broadcasted_iota(jnp.int32, (1, 320), 1)
        lt320 = jnp.zeros((R, 320), f32)
        for j in range(4):
            ltxj = jax.lax.dot_general(
                P128, rlt_ref[:, j, :], (((1,), (0,)), ((), ())),
                preferred_element_type=f32)
            ltj = jnp.sum(jnp.where(lane_sel, ltxj, 0.0),
                          axis=1, keepdims=True)
            lt320 = lt320 + ltj * (lanes320 % 4 == j).astype(f32)

        sel = ((lanes320 // 4).astype(f32) == idx).astype(f32)
        sl1r = _smooth_l1(rlp_ref[...] - lt320)
        loc_sum = jnp.sum(sl1r * sel * pos, keepdims=True)

        x = rcp_ref[...]
        m = jnp.max(x, axis=1, keepdims=True)
        lse = jnp.log(jnp.sum(jnp.exp(x - m), axis=1, keepdims=True)) + m
        lanes81 = jax.lax.broadcasted_iota(jnp.int32, (1, 81), 1).astype(f32)
        pick = jnp.sum(jnp.where(lanes81 == ct, x, 0.0), axis=1, keepdims=True)
        ce_sum = jnp.sum(lse - pick, keepdims=True)

        np_rcnn = jnp.maximum(jnp.sum(pos, keepdims=True), 1.0)
        acc_ref[3:4, 0:1] = (loc_sum + ce_sum) / np_rcnn

    @pl.when(step == nsteps - 1)
    def _fin():
        s_loc = jnp.sum(acc_ref[0:1, :], axis=1, keepdims=True)
        s_ce = jnp.sum(acc_ref[1:2, :], axis=1, keepdims=True)
        np_rpn = jnp.maximum(jnp.sum(acc_ref[2:3, :], axis=1, keepdims=True), 1.0)
        out_ref[...] = (s_loc + s_ce) / np_rpn + acc_ref[3:4, 0:1]


@jax.jit
def kernel(loc_p, cls_p, loc_t, cls_t, rpn_loc_p, rpn_cls_p, rpn_loc_t,
           rpn_cls_t, ignore):
    A = rpn_loc_p.shape[0]
    R = loc_p.shape[0]
    rows = A // 128
    B = 1024
    nsteps = rows // B

    lp = _sublane_view(rpn_loc_p)
    lt = _sublane_view(rpn_loc_t)
    cp = _sublane_view(rpn_cls_p)
    t2 = rpn_cls_t.reshape(rows, 128)
    ign = ignore.reshape(rows, 128).astype(jnp.float32)
    rlt = _sublane_view(loc_t)
    rct = cls_t.reshape(R // 128, 128)

    out = pl.pallas_call(
        _loss_kernel,
        grid=(nsteps,),
        in_specs=[
            pl.BlockSpec((B, 4, 128), lambda i: (i, 0, 0)),
            pl.BlockSpec((B, 4, 128), lambda i: (i, 0, 0)),
            pl.BlockSpec((B, 2, 128), lambda i: (i, 0, 0)),
            pl.BlockSpec((B, 128), lambda i: (i, 0)),
            pl.BlockSpec((B, 128), lambda i: (i, 0)),
            pl.BlockSpec((R, 320), lambda i: (0, 0)),
            pl.BlockSpec((R, 81), lambda i: (0, 0)),
            pl.BlockSpec((R // 128, 4, 128), lambda i: (0, 0, 0)),
            pl.BlockSpec((R // 128, 128), lambda i: (0, 0)),
        ],
        out_specs=pl.BlockSpec((1, 1), lambda i: (0, 0)),
        out_shape=jax.ShapeDtypeStruct((1, 1), jnp.float32),
        scratch_shapes=[pltpu.VMEM((8, 128), jnp.float32)],
    )(lp, lt, cp, t2, ign, loc_p, cls_p, rlt, rct)
    return out[0, 0]
